# baseline (speedup 1.0000x reference)
import functools

import jax
import jax.numpy as jnp
from jax import lax
from jax.experimental import pallas as pl
from jax.experimental.pallas import tpu as pltpu
from jax.experimental.pallas import tpu_sc as plsc

B, N, M = 32, 512, 512
L = 16
NC, NS = 2, 16
ROWS_BLK = 64
SPLITS = ((0, 384), (384, 128))
TC_BATCH = 8
BIG = 2**30


N_GRP = 2
PER_GRP = M // L // N_GRP


def _argmax_merge(accs):
    while len(accs) > 1:
        nxt = []
        for (av, ai), (bv, bi) in zip(accs[::2], accs[1::2]):
            gt = bv > av
            nxt.append((jnp.where(gt, bv, av), jnp.where(gt, bi, ai)))
        accs = nxt
    return accs[0]


def _argmax_extract(best_v, best_i):
    mx = jnp.max(best_v)
    cand = jnp.where(best_v == mx, best_i, jnp.int32(BIG))
    return jnp.min(cand)


def _masked_argmax(row_ref, r, bits, lane_iota):
    accs = []
    for g in range(N_GRP):
        bv = jnp.full((L,), -jnp.inf, jnp.float32)
        bi = jnp.zeros((L,), jnp.int32)
        for k in range(PER_GRP):
            kk = g * PER_GRP + k
            mk = (bits & jnp.int32(((1 << kk) & 0xFFFFFFFF) - (1 << 32 if kk == 31 else 0))) != 0
            v = row_ref[r, pl.ds(kk * L, L)]
            gt = (v > bv) & mk
            bv = jnp.where(gt, v, bv)
            bi = jnp.where(gt, lane_iota + (kk * L), bi)
        accs.append((bv, bi))
    return _argmax_extract(*_argmax_merge(accs))


def _clear_bit(bits, idx, lane_iota):
    bitv = jnp.where(lane_iota == (idx & (L - 1)),
                     jnp.left_shift(jnp.int32(1), idx >> 4), 0)
    return bits & ~bitv


def _write_rem(rem, idx, i, lane_iota):
    idxv = jnp.full((L,), idx, jnp.int32)
    plsc.store_scatter(
        rem, [idxv], jnp.full((L,), i, jnp.int32), mask=lane_iota == 0)


def _p1_rows(rowbuf, rem, row0, lane_iota, bits):
    def pair_body(p, bits):
        ra = 2 * p
        rb = ra + 1
        accs_a = []
        accs_b = []
        for g in range(N_GRP):
            bva = jnp.full((L,), -jnp.inf, jnp.float32)
            bia = jnp.zeros((L,), jnp.int32)
            bvb = jnp.full((L,), -jnp.inf, jnp.float32)
            bib = jnp.zeros((L,), jnp.int32)
            for k in range(PER_GRP):
                kk = g * PER_GRP + k
                mk = (bits & jnp.int32(((1 << kk) & 0xFFFFFFFF) - (1 << 32 if kk == 31 else 0))) != 0
                ii = lane_iota + (kk * L)
                va = rowbuf[ra, pl.ds(kk * L, L)]
                vb = rowbuf[rb, pl.ds(kk * L, L)]
                ga = (va > bva) & mk
                bva = jnp.where(ga, va, bva)
                bia = jnp.where(ga, ii, bia)
                gb = (vb > bvb) & mk
                bvb = jnp.where(gb, vb, bvb)
                bib = jnp.where(gb, ii, bib)
            accs_a.append((bva, bia))
            accs_b.append((bvb, bib))
        idx_a = _argmax_extract(*_argmax_merge(accs_a))
        idx_b = _argmax_extract(*_argmax_merge(accs_b))
        bits = _clear_bit(bits, idx_a, lane_iota)
        _write_rem(rem, idx_a, row0 + ra, lane_iota)

        idx_b = lax.cond(
            idx_b == idx_a,
            lambda: _masked_argmax(rowbuf, rb, bits, lane_iota),
            lambda: idx_b)
        bits = _clear_bit(bits, idx_b, lane_iota)
        _write_rem(rem, idx_b, row0 + rb, lane_iota)
        return bits

    return lax.fori_loop(0, ROWS_BLK // 2, pair_body, bits)


def _p1_block_body(blk, row_base, n_rows):
    def inner(s_hbm, bits_in, rem_in, bits_out, rem_out,
              buf0, buf1, bits_v, rem, sem0, sem1):
        b = lax.axis_index("s") * NC + lax.axis_index("c")
        lane_iota = lax.broadcasted_iota(jnp.int32, (L,), 0)

        if blk == 0:
            bits = jnp.full((L,), -1, jnp.int32)
            for k in range(M // L):
                rem[pl.ds(k * L, L)] = jnp.full((L,), BIG, jnp.int32)
        else:
            pltpu.sync_copy(bits_in.at[b, 0], bits_v)
            pltpu.sync_copy(rem_in.at[b, 0], rem)
            bits = bits_v[...]

        bufs = (buf0, buf1)
        sems = (sem0, sem1)
        n_sub = n_rows // ROWS_BLK
        copies = [None] * n_sub
        copies[0] = pltpu.async_copy(
            s_hbm.at[b, pl.ds(row_base, ROWS_BLK)], bufs[0], sems[0])
        for sub in range(n_sub):
            copies[sub].wait()
            if sub + 1 < n_sub:
                copies[sub + 1] = pltpu.async_copy(
                    s_hbm.at[b, pl.ds(row_base + (sub + 1) * ROWS_BLK,
                                      ROWS_BLK)],
                    bufs[(sub + 1) % 2], sems[(sub + 1) % 2])
            bits = _p1_rows(bufs[sub % 2], rem,
                            row_base + sub * ROWS_BLK, lane_iota, bits)

        bits_v[...] = bits
        pltpu.sync_copy(bits_v, bits_out.at[b, 0])
        pltpu.sync_copy(rem, rem_out.at[b, 0])

    if blk == 0:
        def body0(s_hbm, bits_out, rem_out,
                  buf0, buf1, bits_v, rem, sem0, sem1):
            inner(s_hbm, None, None, bits_out, rem_out,
                  buf0, buf1, bits_v, rem, sem0, sem1)
        return body0
    return inner


def _p1_block(blk, row_base, n_rows, s, pen_state, rem_state):
    mesh = plsc.VectorSubcoreMesh(core_axis_name="c", subcore_axis_name="s")
    kern = functools.partial(
        pl.kernel,
        mesh=mesh,
        out_type=(
            jax.ShapeDtypeStruct((B, 1, L), jnp.int32),
            jax.ShapeDtypeStruct((B, 1, M), jnp.int32),
        ),
        scratch_types=[
            pltpu.VMEM((ROWS_BLK, M), jnp.float32),
            pltpu.VMEM((ROWS_BLK, M), jnp.float32),
            pltpu.VMEM((L,), jnp.int32),
            pltpu.VMEM((M,), jnp.int32),
            pltpu.SemaphoreType.DMA,
            pltpu.SemaphoreType.DMA,
        ],
        compiler_params=pltpu.CompilerParams(needs_layout_passes=False),
        name=f"p1_blk{blk}",
    )(_p1_block_body(blk, row_base, n_rows))
    if blk == 0:
        return kern(s)
    return kern(s, pen_state, rem_state)


def _p2_kernel_body(blk, row0, n_rows, *refs):
    if blk == 0:
        s_ref, rem_ref, o_ref = refs
    else:
        _, s_ref, rem_ref, o_ref = refs
    rows = s_ref[...]
    ra = rem_ref[...]
    row_ids = row0 + lax.broadcasted_iota(jnp.int32, (1, n_rows, 1), 1)
    mask = ra >= row_ids
    neg = jnp.where(mask, rows, -jnp.inf)
    mx = jnp.max(neg, axis=2, keepdims=True)
    e = jnp.exp(neg - mx)
    o_ref[...] = e / jnp.sum(e, axis=2, keepdims=True)


def _p2_block(blk, row0, n_rows, out_prev, s, rem3):
    assert row0 % n_rows == 0
    roff = row0 // n_rows
    blk_spec = pl.BlockSpec(
        (TC_BATCH, n_rows, M), lambda bb: (bb, roff, 0))
    in_specs = [
        blk_spec,
        pl.BlockSpec((TC_BATCH, 1, M), lambda bb: (bb, 0, 0)),
    ]
    operands = (s, rem3)
    aliases = {}
    if blk > 0:
        in_specs = [pl.BlockSpec(memory_space=pl.ANY)] + in_specs
        operands = (out_prev,) + operands
        aliases = {0: 0}
    return pl.pallas_call(
        functools.partial(_p2_kernel_body, blk, row0, n_rows),
        grid=(B // TC_BATCH,),
        in_specs=in_specs,
        out_specs=blk_spec,
        out_shape=jax.ShapeDtypeStruct((B, N, M), jnp.float32),
        input_output_aliases=aliases,
        name=f"p2_blk{blk}",
    )(*operands)


def kernel(s):
    pen_state = rem_state = None
    out = None
    for blk, (row0, n_rows) in enumerate(SPLITS):
        pen_state, rem_state = _p1_block(
            blk, row0, n_rows, s, pen_state, rem_state)
        out = _p2_block(blk, row0, n_rows, out, s, rem_state)
    return out


# speedup vs baseline: 1.0235x; 1.0235x over previous
import functools

import jax
import jax.numpy as jnp
from jax import lax
from jax.experimental import pallas as pl
from jax.experimental.pallas import tpu as pltpu
from jax.experimental.pallas import tpu_sc as plsc

B, N, M = 32, 512, 512
L = 16
NC, NS = 2, 16
ROWS_BLK = 64
SPLITS = ((0, 384), (384, 128))
TC_BATCH = 8
BIG = 2**30


N_GRP = 2
PER_GRP = M // L // N_GRP


def _argmax_merge(accs):
    while len(accs) > 1:
        nxt = []
        for (av, ai), (bv, bi) in zip(accs[::2], accs[1::2]):
            gt = bv > av
            nxt.append((jnp.where(gt, bv, av), jnp.where(gt, bi, ai)))
        accs = nxt
    return accs[0]


def _argmax_extract(best_v, best_i):
    mx = jnp.max(best_v)
    cand = jnp.where(best_v == mx, best_i, jnp.int32(BIG))
    return jnp.min(cand)


def _masked_argmax(row_ref, r, pen, lane_iota):
    accs = []
    for g in range(N_GRP):
        bv = jnp.full((L,), -jnp.inf, jnp.float32)
        bi = jnp.zeros((L,), jnp.int32)
        for k in range(PER_GRP):
            kk = g * PER_GRP + k
            v = row_ref[r, pl.ds(kk * L, L)] + pen[pl.ds(kk * L, L)]
            gt = v > bv
            bv = jnp.where(gt, v, bv)
            bi = jnp.where(gt, lane_iota + (kk * L), bi)
        accs.append((bv, bi))
    return _argmax_extract(*_argmax_merge(accs))


def _remove(pen, rem, idx, i, lane_iota):
    idxv = jnp.full((L,), idx, jnp.int32)
    lane0 = lane_iota == 0
    plsc.store_scatter(
        pen, [idxv], jnp.full((L,), -jnp.inf, jnp.float32), mask=lane0)
    plsc.store_scatter(
        rem, [idxv], jnp.full((L,), i, jnp.int32), mask=lane0)


def _p1_rows(rowbuf, pen, rem, row0, lane_iota):
    def do_pair(ra):
        rb = ra + 1
        accs_a = []
        accs_b = []
        for g in range(N_GRP):
            bva = jnp.full((L,), -jnp.inf, jnp.float32)
            bia = jnp.zeros((L,), jnp.int32)
            bvb = jnp.full((L,), -jnp.inf, jnp.float32)
            bib = jnp.zeros((L,), jnp.int32)
            for k in range(PER_GRP):
                kk = g * PER_GRP + k
                pk = pen[pl.ds(kk * L, L)]
                ii = lane_iota + (kk * L)
                va = rowbuf[ra, pl.ds(kk * L, L)] + pk
                vb = rowbuf[rb, pl.ds(kk * L, L)] + pk
                ga = va > bva
                bva = jnp.where(ga, va, bva)
                bia = jnp.where(ga, ii, bia)
                gb = vb > bvb
                bvb = jnp.where(gb, vb, bvb)
                bib = jnp.where(gb, ii, bib)
            accs_a.append((bva, bia))
            accs_b.append((bvb, bib))
        idx_a = _argmax_extract(*_argmax_merge(accs_a))
        idx_b = _argmax_extract(*_argmax_merge(accs_b))
        _remove(pen, rem, idx_a, row0 + ra, lane_iota)

        @pl.when(idx_b == idx_a)
        def _miss():
            idx_b2 = _masked_argmax(rowbuf, rb, pen, lane_iota)
            _remove(pen, rem, idx_b2, row0 + rb, lane_iota)

        @pl.when(idx_b != idx_a)
        def _hit():
            _remove(pen, rem, idx_b, row0 + rb, lane_iota)

    def pair_body(p, carry):
        do_pair(2 * p)
        return carry

    lax.fori_loop(0, ROWS_BLK // 2, pair_body, 0)


def _p1_block_body(blk, row_base, n_rows):
    def inner(s_hbm, pen_in, rem_in, pen_out, rem_out,
              buf0, buf1, pen, rem, sem0, sem1):
        b = lax.axis_index("s") * NC + lax.axis_index("c")
        lane_iota = lax.broadcasted_iota(jnp.int32, (L,), 0)

        if blk == 0:
            for k in range(M // L):
                pen[pl.ds(k * L, L)] = jnp.zeros((L,), jnp.float32)
                rem[pl.ds(k * L, L)] = jnp.full((L,), BIG, jnp.int32)
        else:
            pltpu.sync_copy(pen_in.at[b, 0], pen)
            pltpu.sync_copy(rem_in.at[b, 0], rem)

        bufs = (buf0, buf1)
        sems = (sem0, sem1)
        n_sub = n_rows // ROWS_BLK
        copies = [None] * n_sub
        copies[0] = pltpu.async_copy(
            s_hbm.at[b, pl.ds(row_base, ROWS_BLK)], bufs[0], sems[0])
        for sub in range(n_sub):
            copies[sub].wait()
            if sub + 1 < n_sub:
                copies[sub + 1] = pltpu.async_copy(
                    s_hbm.at[b, pl.ds(row_base + (sub + 1) * ROWS_BLK,
                                      ROWS_BLK)],
                    bufs[(sub + 1) % 2], sems[(sub + 1) % 2])
            _p1_rows(bufs[sub % 2], pen, rem,
                     row_base + sub * ROWS_BLK, lane_iota)

        pltpu.sync_copy(pen, pen_out.at[b, 0])
        pltpu.sync_copy(rem, rem_out.at[b, 0])

    if blk == 0:
        def body0(s_hbm, pen_out, rem_out,
                  buf0, buf1, pen, rem, sem0, sem1):
            inner(s_hbm, None, None, pen_out, rem_out,
                  buf0, buf1, pen, rem, sem0, sem1)
        return body0
    return inner


def _p1_block(blk, row_base, n_rows, s, pen_state, rem_state):
    mesh = plsc.VectorSubcoreMesh(core_axis_name="c", subcore_axis_name="s")
    kern = functools.partial(
        pl.kernel,
        mesh=mesh,
        out_type=(
            jax.ShapeDtypeStruct((B, 1, M), jnp.float32),
            jax.ShapeDtypeStruct((B, 1, M), jnp.int32),
        ),
        scratch_types=[
            pltpu.VMEM((ROWS_BLK, M), jnp.float32),
            pltpu.VMEM((ROWS_BLK, M), jnp.float32),
            pltpu.VMEM((M,), jnp.float32),
            pltpu.VMEM((M,), jnp.int32),
            pltpu.SemaphoreType.DMA,
            pltpu.SemaphoreType.DMA,
        ],
        compiler_params=pltpu.CompilerParams(needs_layout_passes=False),
        name=f"p1_blk{blk}",
    )(_p1_block_body(blk, row_base, n_rows))
    if blk == 0:
        return kern(s)
    return kern(s, pen_state, rem_state)


def _p2_kernel_body(blk, row0, n_rows, *refs):
    if blk == 0:
        s_ref, rem_ref, o_ref = refs
    else:
        _, s_ref, rem_ref, o_ref = refs
    rows = s_ref[...]
    ra = rem_ref[...]
    row_ids = row0 + lax.broadcasted_iota(jnp.int32, (1, n_rows, 1), 1)
    mask = ra >= row_ids
    neg = jnp.where(mask, rows, -jnp.inf)
    mx = jnp.max(neg, axis=2, keepdims=True)
    e = jnp.exp(neg - mx)
    o_ref[...] = e / jnp.sum(e, axis=2, keepdims=True)


def _p2_block(blk, row0, n_rows, out_prev, s, rem3):
    assert row0 % n_rows == 0
    roff = row0 // n_rows
    blk_spec = pl.BlockSpec(
        (TC_BATCH, n_rows, M), lambda bb: (bb, roff, 0))
    in_specs = [
        blk_spec,
        pl.BlockSpec((TC_BATCH, 1, M), lambda bb: (bb, 0, 0)),
    ]
    operands = (s, rem3)
    aliases = {}
    if blk > 0:
        in_specs = [pl.BlockSpec(memory_space=pl.ANY)] + in_specs
        operands = (out_prev,) + operands
        aliases = {0: 0}
    return pl.pallas_call(
        functools.partial(_p2_kernel_body, blk, row0, n_rows),
        grid=(B // TC_BATCH,),
        in_specs=in_specs,
        out_specs=blk_spec,
        out_shape=jax.ShapeDtypeStruct((B, N, M), jnp.float32),
        input_output_aliases=aliases,
        name=f"p2_blk{blk}",
    )(*operands)


def kernel(s):
    pen_state = rem_state = None
    out = None
    for blk, (row0, n_rows) in enumerate(SPLITS):
        pen_state, rem_state = _p1_block(
            blk, row0, n_rows, s, pen_state, rem_state)
        out = _p2_block(blk, row0, n_rows, out, s, rem_state)
    return out


# speedup vs baseline: 1.0458x; 1.0218x over previous
import functools

import jax
import jax.numpy as jnp
from jax import lax
from jax.experimental import pallas as pl
from jax.experimental.pallas import tpu as pltpu
from jax.experimental.pallas import tpu_sc as plsc

B, N, M = 32, 512, 512
L = 16
NC, NS = 2, 16
ROWS_BLK = 64
SPLITS = ((0, 384), (384, 128))
TC_BATCH = 8
BIG = 2**30


N_GRP = 2
PER_GRP = M // L // N_GRP


def _argmax_merge(accs):
    while len(accs) > 1:
        nxt = []
        for (av, ai), (bv, bi) in zip(accs[::2], accs[1::2]):
            gt = bv > av
            nxt.append((jnp.where(gt, bv, av), jnp.where(gt, bi, ai)))
        accs = nxt
    return accs[0]


def _argmax_extract(best_v, best_i):
    mx = jnp.max(best_v)
    cand = jnp.where(best_v == mx, best_i, jnp.int32(BIG))
    return jnp.min(cand)


def _masked_argmax(row_ref, r, pen, lane_iota):
    accs = []
    for g in range(N_GRP):
        bv = jnp.full((L,), -jnp.inf, jnp.float32)
        bi = jnp.zeros((L,), jnp.int32)
        for k in range(PER_GRP):
            kk = g * PER_GRP + k
            v = row_ref[r, pl.ds(kk * L, L)] + pen[pl.ds(kk * L, L)]
            gt = v > bv
            bv = jnp.where(gt, v, bv)
            bi = jnp.where(gt, lane_iota + (kk * L), bi)
        accs.append((bv, bi))
    return _argmax_extract(*_argmax_merge(accs))


def _remove(pen, rem, idx, i, lane_iota):
    idxv = jnp.full((L,), idx, jnp.int32)
    lane0 = lane_iota == 0
    plsc.store_scatter(
        pen, [idxv], jnp.full((L,), -jnp.inf, jnp.float32), mask=lane0)
    plsc.store_scatter(
        rem, [idxv], jnp.full((L,), i, jnp.int32), mask=lane0)


def _p1_rows(rowbuf, pen, rem, row0, lane_iota):
    def do_pair(ra):
        rb = ra + 1
        accs_a = []
        accs_b = []
        for g in range(N_GRP):
            bva = jnp.full((L,), -jnp.inf, jnp.float32)
            bia = jnp.zeros((L,), jnp.int32)
            bvb = jnp.full((L,), -jnp.inf, jnp.float32)
            bib = jnp.zeros((L,), jnp.int32)
            for k in range(PER_GRP):
                kk = g * PER_GRP + k
                pk = pen[pl.ds(kk * L, L)]
                ii = lane_iota + (kk * L)
                va = rowbuf[ra, pl.ds(kk * L, L)] + pk
                vb = rowbuf[rb, pl.ds(kk * L, L)] + pk
                ga = va > bva
                bva = jnp.where(ga, va, bva)
                bia = jnp.where(ga, ii, bia)
                gb = vb > bvb
                bvb = jnp.where(gb, vb, bvb)
                bib = jnp.where(gb, ii, bib)
            accs_a.append((bva, bia))
            accs_b.append((bvb, bib))
        idx_a = _argmax_extract(*_argmax_merge(accs_a))
        idx_b = _argmax_extract(*_argmax_merge(accs_b))
        _remove(pen, rem, idx_a, row0 + ra, lane_iota)

        @pl.when(idx_b == idx_a)
        def _miss():
            idx_b2 = _masked_argmax(rowbuf, rb, pen, lane_iota)
            _remove(pen, rem, idx_b2, row0 + rb, lane_iota)

        @pl.when(idx_b != idx_a)
        def _hit():
            _remove(pen, rem, idx_b, row0 + rb, lane_iota)

    def pair_body(p, carry):
        do_pair(2 * p)
        return carry

    lax.fori_loop(0, ROWS_BLK // 2, pair_body, 0)


def _p1_block_body(blk, row_base, n_rows):
    def inner(s_hbm, pen_in, rem_in, pen_out, rem_out,
              buf0, buf1, pen, rem, sem0, sem1):
        b = lax.axis_index("s") * NC + lax.axis_index("c")
        lane_iota = lax.broadcasted_iota(jnp.int32, (L,), 0)

        if blk == 0:
            for k in range(M // L):
                pen[pl.ds(k * L, L)] = jnp.zeros((L,), jnp.float32)
                rem[pl.ds(k * L, L)] = jnp.full((L,), BIG, jnp.int32)
        else:
            pltpu.sync_copy(pen_in.at[b, 0], pen)
            pltpu.sync_copy(rem_in.at[b, 0], rem)

        bufs = (buf0, buf1)
        sems = (sem0, sem1)
        n_sub = n_rows // ROWS_BLK
        copies = [None] * n_sub
        copies[0] = pltpu.async_copy(
            s_hbm.at[b, pl.ds(row_base, ROWS_BLK)], bufs[0], sems[0])
        for sub in range(n_sub):
            copies[sub].wait()
            if sub + 1 < n_sub:
                copies[sub + 1] = pltpu.async_copy(
                    s_hbm.at[b, pl.ds(row_base + (sub + 1) * ROWS_BLK,
                                      ROWS_BLK)],
                    bufs[(sub + 1) % 2], sems[(sub + 1) % 2])
            _p1_rows(bufs[sub % 2], pen, rem,
                     row_base + sub * ROWS_BLK, lane_iota)

        pltpu.sync_copy(pen, pen_out.at[b, 0])
        pltpu.sync_copy(rem, rem_out.at[b, 0])

    if blk == 0:
        def body0(s_hbm, pen_out, rem_out,
                  buf0, buf1, pen, rem, sem0, sem1):
            inner(s_hbm, None, None, pen_out, rem_out,
                  buf0, buf1, pen, rem, sem0, sem1)
        return body0
    return inner


def _p1_block(blk, row_base, n_rows, s, pen_state, rem_state):
    mesh = plsc.VectorSubcoreMesh(core_axis_name="c", subcore_axis_name="s")
    kern = functools.partial(
        pl.kernel,
        mesh=mesh,
        out_type=(
            jax.ShapeDtypeStruct((B, 1, M), jnp.float32),
            jax.ShapeDtypeStruct((B, 1, M), jnp.int32),
        ),
        scratch_types=[
            pltpu.VMEM((ROWS_BLK, M), jnp.float32),
            pltpu.VMEM((ROWS_BLK, M), jnp.float32),
            pltpu.VMEM((M,), jnp.float32),
            pltpu.VMEM((M,), jnp.int32),
            pltpu.SemaphoreType.DMA,
            pltpu.SemaphoreType.DMA,
        ],
        compiler_params=pltpu.CompilerParams(needs_layout_passes=False),
        name=f"p1_blk{blk}",
    )(_p1_block_body(blk, row_base, n_rows))
    if blk == 0:
        return kern(s)
    return kern(s, pen_state, rem_state)


def _p2_kernel_body(blk, row0, n_rows, *refs):
    if blk == 0:
        s_ref, rem_ref, o_ref = refs
    else:
        _, s_ref, rem_ref, o_ref = refs
    rows = s_ref[...]
    ra = rem_ref[...]
    row_ids = row0 + lax.broadcasted_iota(jnp.int32, (1, n_rows, 1), 1)
    mask = ra >= row_ids
    neg = jnp.where(mask, rows, -jnp.inf)
    mx = jnp.max(neg, axis=2, keepdims=True)
    e = jnp.exp(neg - mx)
    o_ref[...] = e / jnp.sum(e, axis=2, keepdims=True)


def _p2_block(blk, row0, n_rows, out_prev, s, rem3):
    assert row0 % n_rows == 0
    roff = row0 // n_rows
    tcb = TC_BATCH if blk == 0 else 16
    blk_spec = pl.BlockSpec(
        (tcb, n_rows, M), lambda bb: (bb, roff, 0))
    in_specs = [
        blk_spec,
        pl.BlockSpec((tcb, 1, M), lambda bb: (bb, 0, 0)),
    ]
    operands = (s, rem3)
    aliases = {}
    if blk > 0:
        in_specs = [pl.BlockSpec(memory_space=pl.ANY)] + in_specs
        operands = (out_prev,) + operands
        aliases = {0: 0}
    return pl.pallas_call(
        functools.partial(_p2_kernel_body, blk, row0, n_rows),
        grid=(B // tcb,),
        in_specs=in_specs,
        out_specs=blk_spec,
        out_shape=jax.ShapeDtypeStruct((B, N, M), jnp.float32),
        input_output_aliases=aliases,
        name=f"p2_blk{blk}",
    )(*operands)


def kernel(s):
    pen_state = rem_state = None
    out = None
    for blk, (row0, n_rows) in enumerate(SPLITS):
        pen_state, rem_state = _p1_block(
            blk, row0, n_rows, s, pen_state, rem_state)
        out = _p2_block(blk, row0, n_rows, out, s, rem_state)
    return out
